# Initial kernel scaffold; baseline (speedup 1.0000x reference)
import jax
import jax.numpy as jnp
from jax.experimental import pallas as pl


def kernel(x, y, edge_index, edge_vals, c, b, W_ve, b_ve, W_ce, b_ce, Ukx_W, Ukx_b, Uky_W, Uky_b, tau, Vky_W, Vky_b, Wkx_W, Wkx_b, Vkx_W, Vkx_b, sigma, W_o1, b_o1, W_o2):
    raise NotImplementedError("write your pallas kernel here")


# speedup vs baseline: 3.7030x; 3.7030x over previous
import functools

import jax
import jax.numpy as jnp
from jax import lax
from jax.experimental import pallas as pl
from jax.experimental.pallas import tpu as pltpu
from jax.experimental.pallas import tpu_sc as plsc

_N = 10000
_D = 256
_H = 128
_NC = 2
_NS = 16
_CH = 128
_NPAD = 10240
_RPT = _NPAD // _NS
_BM = 1000
_GRID = _N // _BM

_HIGH = jax.lax.Precision.HIGHEST


def _spmm_body(ept, mat0, mat1, src, dst, vals, out0, out1,
               acc, rbuf, zbuf, idx_s, idx_d, vbuf, sem):
    c = lax.axis_index("c")
    s = lax.axis_index("s")

    zv = jnp.zeros((16,), jnp.float32)

    def zrow(i, carry):
        for j in range(8):
            zbuf[i, pl.ds(j * 16, 16)] = zv
        return carry

    lax.fori_loop(0, _CH, zrow, 0)
    r0 = s * _RPT
    for k in range(_RPT // _CH):
        pltpu.sync_copy(zbuf, acc.at[pl.ds(r0 + k * _CH, _CH)])
    rem = _RPT % _CH
    if rem:
        pltpu.sync_copy(zbuf.at[pl.ds(0, rem)],
                        acc.at[pl.ds(r0 + (_RPT // _CH) * _CH, rem)])
    plsc.subcore_barrier()

    base0 = s * ept

    def chunk(i, carry):
        eb = base0 + i * _CH
        pltpu.sync_copy(src.at[pl.ds(eb, _CH)], idx_s)
        pltpu.sync_copy(dst.at[pl.ds(eb, _CH)], idx_d)
        pltpu.sync_copy(vals.at[pl.ds(eb, _CH)], vbuf)

        @pl.when(c == 0)
        def _():
            pltpu.async_copy(mat0.at[idx_s], rbuf, sem).wait()

        @pl.when(c == 1)
        def _():
            pltpu.async_copy(mat1.at[idx_s], rbuf, sem).wait()

        def grp(k, rcarry):
            vv = vbuf[pl.ds(k * 16, 16)]
            for i in range(16):
                bv = jnp.take_along_axis(
                    vv, jnp.full((16,), i, jnp.int32), axis=0)
                e = k * 16 + i
                for j in range(8):
                    sl = pl.ds(j * 16, 16)
                    rbuf[e, sl] = rbuf[e, sl] * bv
            return rcarry

        lax.fori_loop(0, _CH // 16, grp, 0)
        pltpu.sync_copy(rbuf, acc.at[idx_d], add=True)
        return carry

    lax.fori_loop(0, ept // _CH, chunk, 0)
    plsc.subcore_barrier()

    @pl.when(c == 0)
    def _():
        pltpu.sync_copy(acc.at[pl.ds(r0, _RPT)], out0.at[pl.ds(r0, _RPT)])

    @pl.when(c == 1)
    def _():
        pltpu.sync_copy(acc.at[pl.ds(r0, _RPT)], out1.at[pl.ds(r0, _RPT)])


@functools.lru_cache(maxsize=None)
def _make_spmm(ept):
    mesh = plsc.VectorSubcoreMesh(core_axis_name="c", subcore_axis_name="s",
                                  num_cores=_NC, num_subcores=_NS)
    return pl.kernel(
        functools.partial(_spmm_body, ept),
        out_type=(jax.ShapeDtypeStruct((_NPAD, _H), jnp.float32),
                  jax.ShapeDtypeStruct((_NPAD, _H), jnp.float32)),
        mesh=mesh,
        scratch_types=[
            pltpu.VMEM_SHARED((_NPAD, _H), jnp.float32),
            pltpu.VMEM((_CH, _H), jnp.float32),
            pltpu.VMEM((_CH, _H), jnp.float32),
            pltpu.VMEM((_CH,), jnp.int32),
            pltpu.VMEM((_CH,), jnp.int32),
            pltpu.VMEM((_CH,), jnp.float32),
            pltpu.SemaphoreType.DMA,
        ],
    )


def _spmm(mat0, mat1, src, dst, vals, ept):
    return _make_spmm(ept)(mat0, mat1, src, dst, vals)


def _embed_body(h_ref, w_ref, b_ref, o0_ref, o1_ref):
    h = h_ref[...]
    w = w_ref[...]
    acc = jnp.dot(h, w, preferred_element_type=jnp.float32,
                  precision=_HIGH) + b_ref[...]
    acc = jnp.maximum(acc, 0.0)
    o0_ref[...] = acc[:, :_H]
    o1_ref[...] = acc[:, _H:]


def _mid_body(h0_ref, h1_ref, w_ref, b_ref, o0_ref, o1_ref):
    w = w_ref[...]
    acc = (jnp.dot(h0_ref[...], w[:_H, :], preferred_element_type=jnp.float32,
                   precision=_HIGH)
           + jnp.dot(h1_ref[...], w[_H:, :], preferred_element_type=jnp.float32,
                     precision=_HIGH)
           + b_ref[...])
    o0_ref[...] = acc[:, :_H]
    o1_ref[...] = acc[:, _H:]


def _mid2_body(h0_ref, h1_ref, w1_ref, b1_ref, w2_ref, b2_ref, o0_ref, o1_ref):
    w = 2.0 * w1_ref[...] - w2_ref[...]
    bb = 2.0 * b1_ref[...] - b2_ref[...]
    acc = (jnp.dot(h0_ref[...], w[:_H, :], preferred_element_type=jnp.float32,
                   precision=_HIGH)
           + jnp.dot(h1_ref[...], w[_H:, :], preferred_element_type=jnp.float32,
                     precision=_HIGH)
           + bb)
    o0_ref[...] = acc[:, :_H]
    o1_ref[...] = acc[:, _H:]


def _update_body(h0_ref, h1_ref, w_ref, b_ref, a0_ref, a1_ref, cv_ref, s_ref,
                 o0_ref, o1_ref):
    w = w_ref[...]
    sc = s_ref[0, 0]
    cv = cv_ref[...]
    acc = (jnp.dot(h0_ref[...], w[:_H, :], preferred_element_type=jnp.float32,
                   precision=_HIGH)
           + jnp.dot(h1_ref[...], w[_H:, :], preferred_element_type=jnp.float32,
                     precision=_HIGH)
           + b_ref[...])
    a = jnp.concatenate([a0_ref[...], a1_ref[...]], axis=1)
    acc = jnp.maximum(acc + sc * (a - cv), 0.0)
    o0_ref[...] = acc[:, :_H]
    o1_ref[...] = acc[:, _H:]


def _head_body(h0_ref, h1_ref, w1_ref, b_ref, w2_ref, o_ref):
    w1 = w1_ref[...]
    t = (jnp.dot(h0_ref[...], w1[:_H, :], preferred_element_type=jnp.float32,
                 precision=_HIGH)
         + jnp.dot(h1_ref[...], w1[_H:, :], preferred_element_type=jnp.float32,
                   precision=_HIGH)
         + b_ref[...])
    t = jnp.maximum(t, 0.0)
    o_ref[...] = jnp.dot(t, w2_ref[...], preferred_element_type=jnp.float32,
                         precision=_HIGH)


def _rows(i):
    return (i, 0)


def _rep(i):
    return (0, 0)


_SPEC_HF = pl.BlockSpec((_BM, _H), _rows)
_SPEC_FULL = pl.BlockSpec((_BM, _D), _rows)
_SPEC_W = pl.BlockSpec((_D, _D), _rep)
_SPEC_B = pl.BlockSpec((1, _D), _rep)
_SPEC_CV = pl.BlockSpec((_BM, 1), _rows)
_SPEC_S = pl.BlockSpec((1, 1), _rep)
_SPEC_W2 = pl.BlockSpec((_D, 1), _rep)
_SPEC_O1 = pl.BlockSpec((_BM, 1), _rows)

_PAIR_OUT = (jax.ShapeDtypeStruct((_N, _H), jnp.float32),
             jax.ShapeDtypeStruct((_N, _H), jnp.float32))
_PAIR_SPEC = (_SPEC_HF, _SPEC_HF)

_embed = pl.pallas_call(
    _embed_body, grid=(_GRID,),
    in_specs=[_SPEC_FULL, _SPEC_W, _SPEC_B],
    out_specs=_PAIR_SPEC, out_shape=_PAIR_OUT)

_mid = pl.pallas_call(
    _mid_body, grid=(_GRID,),
    in_specs=[_SPEC_HF, _SPEC_HF, _SPEC_W, _SPEC_B],
    out_specs=_PAIR_SPEC, out_shape=_PAIR_OUT)

_mid2 = pl.pallas_call(
    _mid2_body, grid=(_GRID,),
    in_specs=[_SPEC_HF, _SPEC_HF, _SPEC_W, _SPEC_B, _SPEC_W, _SPEC_B],
    out_specs=_PAIR_SPEC, out_shape=_PAIR_OUT)

_update = pl.pallas_call(
    _update_body, grid=(_GRID,),
    in_specs=[_SPEC_HF, _SPEC_HF, _SPEC_W, _SPEC_B, _SPEC_HF, _SPEC_HF,
              _SPEC_CV, _SPEC_S],
    out_specs=_PAIR_SPEC, out_shape=_PAIR_OUT)

_head = pl.pallas_call(
    _head_body, grid=(_GRID,),
    in_specs=[_SPEC_HF, _SPEC_HF, _SPEC_W, _SPEC_B, _SPEC_W2],
    out_specs=_SPEC_O1,
    out_shape=jax.ShapeDtypeStruct((_N, 1), jnp.float32))


def kernel(x, y, edge_index, edge_vals, c, b, W_ve, b_ve, W_ce, b_ce,
           Ukx_W, Ukx_b, Uky_W, Uky_b, tau,
           Vky_W, Vky_b, Wkx_W, Wkx_b, Vkx_W, Vkx_b, sigma,
           W_o1, b_o1, W_o2):
    L = Ukx_W.shape[0]
    rows = edge_index[0].astype(jnp.int32)
    cols = edge_index[1].astype(jnp.int32)
    vals = edge_vals.astype(jnp.float32)

    E = vals.shape[0]
    ept = -(-E // (_NS * _CH)) * _CH
    e_pad = ept * _NS
    if e_pad != E:
        pz = e_pad - E
        rows = jnp.concatenate([rows, jnp.zeros((pz,), jnp.int32)])
        cols = jnp.concatenate([cols, jnp.zeros((pz,), jnp.int32)])
        vals = jnp.concatenate([vals, jnp.zeros((pz,), jnp.float32)])

    b_ve2 = b_ve.reshape(1, _D)
    b_ce2 = b_ce.reshape(1, _D)
    b_o12 = b_o1.reshape(1, _D)

    x0, x1 = _embed(x, W_ve, b_ve2)
    y0, y1 = _embed(y, W_ce, b_ce2)

    for l in range(L):
        u0, u1 = _mid(y0, y1, Uky_W[l], Uky_b[l].reshape(1, _D))
        a0, a1 = _spmm(u0, u1, rows, cols, vals, ept)
        x0, x1 = _update(x0, x1, Ukx_W[l], Ukx_b[l].reshape(1, _D),
                         a0, a1, c, tau[l].reshape(1, 1))
        z0, z1 = _mid2(x0, x1, Wkx_W[l], Wkx_b[l].reshape(1, _D),
                       Vkx_W[l], Vkx_b[l].reshape(1, _D))
        s0, s1 = _spmm(z0, z1, cols, rows, vals, ept)
        y0, y1 = _update(y0, y1, Vky_W[l], Vky_b[l].reshape(1, _D),
                         s0, s1, b, sigma[l].reshape(1, 1))

    x_out = _head(x0, x1, W_o1, b_o12, W_o2).reshape(_N)
    y_out = _head(y0, y1, W_o1, b_o12, W_o2).reshape(_N)
    return (x_out, y_out)


# speedup vs baseline: 4.4374x; 1.1983x over previous
import functools

import jax
import jax.numpy as jnp
from jax import lax
from jax.experimental import pallas as pl
from jax.experimental.pallas import tpu as pltpu
from jax.experimental.pallas import tpu_sc as plsc

_N = 10000
_D = 256
_H = 128
_NC = 2
_NS = 16
_CH = 128
_SB = 16
_NPAD = 10240
_RPT = _NPAD // _NS
_BM = 1000
_GRID = _N // _BM

_HIGH = jax.lax.Precision.HIGHEST


def _scale_chunk(rbuf, vbuf, i):
    def grp(k, rcarry):
        vv = vbuf[i, pl.ds(k * 16, 16)]
        for t in range(16):
            bv = jnp.take_along_axis(
                vv, jnp.full((16,), t, jnp.int32), axis=0)
            e = k * 16 + t
            for j in range(8):
                sl = pl.ds(j * 16, 16)
                rbuf[e, sl] = rbuf[e, sl] * bv
        return rcarry

    lax.fori_loop(0, _CH // 16, grp, 0)


def _spmm_body(nchunk, mat0, mat1, src, dst, vals, out0, out1,
               acc, rbuf0, rbuf1, sidx, didx, vbuf, sem0, sem1):
    c = lax.axis_index("c")
    s = lax.axis_index("s")

    zv = jnp.zeros((16,), jnp.float32)

    def zrow(i, carry):
        for j in range(8):
            rbuf0[i, pl.ds(j * 16, 16)] = zv
        return carry

    lax.fori_loop(0, _CH, zrow, 0)
    r0 = s * _RPT
    for k in range(_RPT // _CH):
        pltpu.sync_copy(rbuf0, acc.at[pl.ds(r0 + k * _CH, _CH)])
    plsc.subcore_barrier()

    def gather(i, rbuf, sem):
        @pl.when(c == 0)
        def _():
            pltpu.async_copy(mat0.at[sidx.at[i]], rbuf, sem)

        @pl.when(c == 1)
        def _():
            pltpu.async_copy(mat1.at[sidx.at[i]], rbuf, sem)

    def wait(i, rbuf, sem):
        @pl.when(c == 0)
        def _():
            pltpu.make_async_copy(mat0.at[sidx.at[i]], rbuf, sem).wait()

        @pl.when(c == 1)
        def _():
            pltpu.make_async_copy(mat1.at[sidx.at[i]], rbuf, sem).wait()

    def sblock(bi, bcarry):
        pltpu.sync_copy(src.at[s, pl.ds(bi * _SB, _SB)], sidx)
        pltpu.sync_copy(dst.at[s, pl.ds(bi * _SB, _SB)], didx)
        pltpu.sync_copy(vals.at[s, pl.ds(bi * _SB, _SB)], vbuf)
        gather(0, rbuf0, sem0)

        def body(j, carry):
            i0 = 2 * j
            i1 = 2 * j + 1
            gather(i1, rbuf1, sem1)
            wait(i0, rbuf0, sem0)
            _scale_chunk(rbuf0, vbuf, i0)
            pltpu.sync_copy(rbuf0, acc.at[didx.at[i0]], add=True)

            @pl.when(j + 1 < _SB // 2)
            def _():
                gather(i1 + 1, rbuf0, sem0)

            wait(i1, rbuf1, sem1)
            _scale_chunk(rbuf1, vbuf, i1)
            pltpu.sync_copy(rbuf1, acc.at[didx.at[i1]], add=True)
            return carry

        lax.fori_loop(0, _SB // 2, body, 0)
        return bcarry

    lax.fori_loop(0, nchunk // _SB, sblock, 0)
    plsc.subcore_barrier()

    @pl.when(c == 0)
    def _():
        pltpu.sync_copy(acc.at[pl.ds(r0, _RPT)], out0.at[pl.ds(r0, _RPT)])

    @pl.when(c == 1)
    def _():
        pltpu.sync_copy(acc.at[pl.ds(r0, _RPT)], out1.at[pl.ds(r0, _RPT)])


@functools.lru_cache(maxsize=None)
def _make_spmm(nchunk):
    mesh = plsc.VectorSubcoreMesh(core_axis_name="c", subcore_axis_name="s",
                                  num_cores=_NC, num_subcores=_NS)
    return pl.kernel(
        functools.partial(_spmm_body, nchunk),
        out_type=(jax.ShapeDtypeStruct((_NPAD, _H), jnp.float32),
                  jax.ShapeDtypeStruct((_NPAD, _H), jnp.float32)),
        mesh=mesh,
        scratch_types=[
            pltpu.VMEM_SHARED((_NPAD, _H), jnp.float32),
            pltpu.VMEM((_CH, _H), jnp.float32),
            pltpu.VMEM((_CH, _H), jnp.float32),
            pltpu.VMEM((_SB, _CH), jnp.int32),
            pltpu.VMEM((_SB, _CH), jnp.int32),
            pltpu.VMEM((_SB, _CH), jnp.float32),
            pltpu.SemaphoreType.DMA,
            pltpu.SemaphoreType.DMA,
        ],
    )


def _spmm(mat0, mat1, src, dst, vals, nchunk):
    return _make_spmm(nchunk)(mat0, mat1, src, dst, vals)


def _embed_body(h_ref, w_ref, b_ref, o0_ref, o1_ref):
    h = h_ref[...]
    w = w_ref[...]
    acc = jnp.dot(h, w, preferred_element_type=jnp.float32,
                  precision=_HIGH) + b_ref[...]
    acc = jnp.maximum(acc, 0.0)
    o0_ref[...] = acc[:, :_H]
    o1_ref[...] = acc[:, _H:]


def _mid_body(h0_ref, h1_ref, w_ref, b_ref, o0_ref, o1_ref):
    w = w_ref[...]
    acc = (jnp.dot(h0_ref[...], w[:_H, :], preferred_element_type=jnp.float32,
                   precision=_HIGH)
           + jnp.dot(h1_ref[...], w[_H:, :], preferred_element_type=jnp.float32,
                     precision=_HIGH)
           + b_ref[...])
    o0_ref[...] = acc[:, :_H]
    o1_ref[...] = acc[:, _H:]


def _mid2_body(h0_ref, h1_ref, w1_ref, b1_ref, w2_ref, b2_ref, o0_ref, o1_ref):
    w = 2.0 * w1_ref[...] - w2_ref[...]
    bb = 2.0 * b1_ref[...] - b2_ref[...]
    acc = (jnp.dot(h0_ref[...], w[:_H, :], preferred_element_type=jnp.float32,
                   precision=_HIGH)
           + jnp.dot(h1_ref[...], w[_H:, :], preferred_element_type=jnp.float32,
                     precision=_HIGH)
           + bb)
    o0_ref[...] = acc[:, :_H]
    o1_ref[...] = acc[:, _H:]


def _update_body(h0_ref, h1_ref, w_ref, b_ref, a0_ref, a1_ref, cv_ref, s_ref,
                 o0_ref, o1_ref):
    w = w_ref[...]
    sc = s_ref[0, 0]
    cv = cv_ref[...]
    acc = (jnp.dot(h0_ref[...], w[:_H, :], preferred_element_type=jnp.float32,
                   precision=_HIGH)
           + jnp.dot(h1_ref[...], w[_H:, :], preferred_element_type=jnp.float32,
                     precision=_HIGH)
           + b_ref[...])
    a = jnp.concatenate([a0_ref[...], a1_ref[...]], axis=1)
    acc = jnp.maximum(acc + sc * (a - cv), 0.0)
    o0_ref[...] = acc[:, :_H]
    o1_ref[...] = acc[:, _H:]


def _head_body(h0_ref, h1_ref, w1_ref, b_ref, w2_ref, o_ref):
    w1 = w1_ref[...]
    t = (jnp.dot(h0_ref[...], w1[:_H, :], preferred_element_type=jnp.float32,
                 precision=_HIGH)
         + jnp.dot(h1_ref[...], w1[_H:, :], preferred_element_type=jnp.float32,
                   precision=_HIGH)
         + b_ref[...])
    t = jnp.maximum(t, 0.0)
    o_ref[...] = jnp.dot(t, w2_ref[...], preferred_element_type=jnp.float32,
                         precision=_HIGH)


def _rows(i):
    return (i, 0)


def _rep(i):
    return (0, 0)


_SPEC_HF = pl.BlockSpec((_BM, _H), _rows)
_SPEC_FULL = pl.BlockSpec((_BM, _D), _rows)
_SPEC_W = pl.BlockSpec((_D, _D), _rep)
_SPEC_B = pl.BlockSpec((1, _D), _rep)
_SPEC_CV = pl.BlockSpec((_BM, 1), _rows)
_SPEC_S = pl.BlockSpec((1, 1), _rep)
_SPEC_W2 = pl.BlockSpec((_D, 1), _rep)
_SPEC_O1 = pl.BlockSpec((_BM, 1), _rows)

_PAIR_OUT = (jax.ShapeDtypeStruct((_N, _H), jnp.float32),
             jax.ShapeDtypeStruct((_N, _H), jnp.float32))
_PAIR_SPEC = (_SPEC_HF, _SPEC_HF)

_embed = pl.pallas_call(
    _embed_body, grid=(_GRID,),
    in_specs=[_SPEC_FULL, _SPEC_W, _SPEC_B],
    out_specs=_PAIR_SPEC, out_shape=_PAIR_OUT)

_mid = pl.pallas_call(
    _mid_body, grid=(_GRID,),
    in_specs=[_SPEC_HF, _SPEC_HF, _SPEC_W, _SPEC_B],
    out_specs=_PAIR_SPEC, out_shape=_PAIR_OUT)

_mid2 = pl.pallas_call(
    _mid2_body, grid=(_GRID,),
    in_specs=[_SPEC_HF, _SPEC_HF, _SPEC_W, _SPEC_B, _SPEC_W, _SPEC_B],
    out_specs=_PAIR_SPEC, out_shape=_PAIR_OUT)

_update = pl.pallas_call(
    _update_body, grid=(_GRID,),
    in_specs=[_SPEC_HF, _SPEC_HF, _SPEC_W, _SPEC_B, _SPEC_HF, _SPEC_HF,
              _SPEC_CV, _SPEC_S],
    out_specs=_PAIR_SPEC, out_shape=_PAIR_OUT)

_head = pl.pallas_call(
    _head_body, grid=(_GRID,),
    in_specs=[_SPEC_HF, _SPEC_HF, _SPEC_W, _SPEC_B, _SPEC_W2],
    out_specs=_SPEC_O1,
    out_shape=jax.ShapeDtypeStruct((_N, 1), jnp.float32))


def kernel(x, y, edge_index, edge_vals, c, b, W_ve, b_ve, W_ce, b_ce,
           Ukx_W, Ukx_b, Uky_W, Uky_b, tau,
           Vky_W, Vky_b, Wkx_W, Wkx_b, Vkx_W, Vkx_b, sigma,
           W_o1, b_o1, W_o2):
    L = Ukx_W.shape[0]
    rows = edge_index[0].astype(jnp.int32)
    cols = edge_index[1].astype(jnp.int32)
    vals = edge_vals.astype(jnp.float32)

    E = vals.shape[0]
    nchunk = -(-E // (_NS * _SB * _CH)) * _SB
    e_pad = nchunk * _CH * _NS
    if e_pad != E:
        pz = e_pad - E
        rows = jnp.concatenate([rows, jnp.zeros((pz,), jnp.int32)])
        cols = jnp.concatenate([cols, jnp.zeros((pz,), jnp.int32)])
        vals = jnp.concatenate([vals, jnp.zeros((pz,), jnp.float32)])
    rows = rows.reshape(_NS, nchunk, _CH)
    cols = cols.reshape(_NS, nchunk, _CH)
    vals = vals.reshape(_NS, nchunk, _CH)

    b_ve2 = b_ve.reshape(1, _D)
    b_ce2 = b_ce.reshape(1, _D)
    b_o12 = b_o1.reshape(1, _D)

    x0, x1 = _embed(x, W_ve, b_ve2)
    y0, y1 = _embed(y, W_ce, b_ce2)

    for l in range(L):
        u0, u1 = _mid(y0, y1, Uky_W[l], Uky_b[l].reshape(1, _D))
        a0, a1 = _spmm(u0, u1, rows, cols, vals, nchunk)
        x0, x1 = _update(x0, x1, Ukx_W[l], Ukx_b[l].reshape(1, _D),
                         a0, a1, c, tau[l].reshape(1, 1))
        z0, z1 = _mid2(x0, x1, Wkx_W[l], Wkx_b[l].reshape(1, _D),
                       Vkx_W[l], Vkx_b[l].reshape(1, _D))
        s0, s1 = _spmm(z0, z1, cols, rows, vals, nchunk)
        y0, y1 = _update(y0, y1, Vky_W[l], Vky_b[l].reshape(1, _D),
                         s0, s1, b, sigma[l].reshape(1, 1))

    x_out = _head(x0, x1, W_o1, b_o12, W_o2).reshape(_N)
    y_out = _head(y0, y1, W_o1, b_o12, W_o2).reshape(_N)
    return (x_out, y_out)


# speedup vs baseline: 4.7295x; 1.0658x over previous
import functools

import jax
import jax.numpy as jnp
from jax import lax
from jax.experimental import pallas as pl
from jax.experimental.pallas import tpu as pltpu
from jax.experimental.pallas import tpu_sc as plsc

_N = 10000
_D = 256
_H = 128
_NC = 2
_NS = 16
_CH = 128
_SB = 16
_NPAD = 10240
_RPT = _NPAD // _NS
_BM = 1000
_GRID = _N // _BM

_HIGH = jax.lax.Precision.DEFAULT


def _scale_chunk(rbuf, vbuf, i):
    def grp(k, rcarry):
        vv = vbuf[i, pl.ds(k * 16, 16)]
        for t in range(16):
            bv = jnp.take_along_axis(
                vv, jnp.full((16,), t, jnp.int32), axis=0)
            e = k * 16 + t
            for j in range(8):
                sl = pl.ds(j * 16, 16)
                rbuf[e, sl] = rbuf[e, sl] * bv
        return rcarry

    lax.fori_loop(0, _CH // 16, grp, 0)


def _spmm_body(nchunk, mat0, mat1, src, dst, vals, out0, out1,
               acc, rbuf0, rbuf1, sidx, didx, vbuf, sem0, sem1):
    c = lax.axis_index("c")
    s = lax.axis_index("s")

    zv = jnp.zeros((16,), jnp.float32)

    def zrow(i, carry):
        for j in range(8):
            rbuf0[i, pl.ds(j * 16, 16)] = zv
        return carry

    lax.fori_loop(0, _CH, zrow, 0)
    r0 = s * _RPT
    for k in range(_RPT // _CH):
        pltpu.sync_copy(rbuf0, acc.at[pl.ds(r0 + k * _CH, _CH)])
    plsc.subcore_barrier()

    def gather(i, rbuf, sem):
        @pl.when(c == 0)
        def _():
            pltpu.async_copy(mat0.at[sidx.at[i]], rbuf, sem)

        @pl.when(c == 1)
        def _():
            pltpu.async_copy(mat1.at[sidx.at[i]], rbuf, sem)

    def wait(i, rbuf, sem):
        @pl.when(c == 0)
        def _():
            pltpu.make_async_copy(mat0.at[sidx.at[i]], rbuf, sem).wait()

        @pl.when(c == 1)
        def _():
            pltpu.make_async_copy(mat1.at[sidx.at[i]], rbuf, sem).wait()

    def sblock(bi, bcarry):
        pltpu.sync_copy(src.at[s, pl.ds(bi * _SB, _SB)], sidx)
        pltpu.sync_copy(dst.at[s, pl.ds(bi * _SB, _SB)], didx)
        pltpu.sync_copy(vals.at[s, pl.ds(bi * _SB, _SB)], vbuf)
        gather(0, rbuf0, sem0)

        def body(j, carry):
            i0 = 2 * j
            i1 = 2 * j + 1
            gather(i1, rbuf1, sem1)
            wait(i0, rbuf0, sem0)
            _scale_chunk(rbuf0, vbuf, i0)
            pltpu.sync_copy(rbuf0, acc.at[didx.at[i0]], add=True)

            @pl.when(j + 1 < _SB // 2)
            def _():
                gather(i1 + 1, rbuf0, sem0)

            wait(i1, rbuf1, sem1)
            _scale_chunk(rbuf1, vbuf, i1)
            pltpu.sync_copy(rbuf1, acc.at[didx.at[i1]], add=True)
            return carry

        lax.fori_loop(0, _SB // 2, body, 0)
        return bcarry

    lax.fori_loop(0, nchunk // _SB, sblock, 0)
    plsc.subcore_barrier()

    @pl.when(c == 0)
    def _():
        pltpu.sync_copy(acc.at[pl.ds(r0, _RPT)], out0.at[pl.ds(r0, _RPT)])

    @pl.when(c == 1)
    def _():
        pltpu.sync_copy(acc.at[pl.ds(r0, _RPT)], out1.at[pl.ds(r0, _RPT)])


@functools.lru_cache(maxsize=None)
def _make_spmm(nchunk):
    mesh = plsc.VectorSubcoreMesh(core_axis_name="c", subcore_axis_name="s",
                                  num_cores=_NC, num_subcores=_NS)
    return pl.kernel(
        functools.partial(_spmm_body, nchunk),
        out_type=(jax.ShapeDtypeStruct((_NPAD, _H), jnp.float32),
                  jax.ShapeDtypeStruct((_NPAD, _H), jnp.float32)),
        mesh=mesh,
        scratch_types=[
            pltpu.VMEM_SHARED((_NPAD, _H), jnp.float32),
            pltpu.VMEM((_CH, _H), jnp.float32),
            pltpu.VMEM((_CH, _H), jnp.float32),
            pltpu.VMEM((_SB, _CH), jnp.int32),
            pltpu.VMEM((_SB, _CH), jnp.int32),
            pltpu.VMEM((_SB, _CH), jnp.float32),
            pltpu.SemaphoreType.DMA,
            pltpu.SemaphoreType.DMA,
        ],
    )


def _spmm(mat0, mat1, src, dst, vals, nchunk):
    return _make_spmm(nchunk)(mat0, mat1, src, dst, vals)


def _embed_body(h_ref, w_ref, b_ref, o0_ref, o1_ref):
    h = h_ref[...]
    w = w_ref[...]
    acc = jnp.dot(h, w, preferred_element_type=jnp.float32,
                  precision=_HIGH) + b_ref[...]
    acc = jnp.maximum(acc, 0.0)
    o0_ref[...] = acc[:, :_H]
    o1_ref[...] = acc[:, _H:]


def _mid_body(h0_ref, h1_ref, w_ref, b_ref, o0_ref, o1_ref):
    w = w_ref[...]
    acc = (jnp.dot(h0_ref[...], w[:_H, :], preferred_element_type=jnp.float32,
                   precision=_HIGH)
           + jnp.dot(h1_ref[...], w[_H:, :], preferred_element_type=jnp.float32,
                     precision=_HIGH)
           + b_ref[...])
    o0_ref[...] = acc[:, :_H]
    o1_ref[...] = acc[:, _H:]


def _mid2_body(h0_ref, h1_ref, w1_ref, b1_ref, w2_ref, b2_ref, o0_ref, o1_ref):
    w = 2.0 * w1_ref[...] - w2_ref[...]
    bb = 2.0 * b1_ref[...] - b2_ref[...]
    acc = (jnp.dot(h0_ref[...], w[:_H, :], preferred_element_type=jnp.float32,
                   precision=_HIGH)
           + jnp.dot(h1_ref[...], w[_H:, :], preferred_element_type=jnp.float32,
                     precision=_HIGH)
           + bb)
    o0_ref[...] = acc[:, :_H]
    o1_ref[...] = acc[:, _H:]


def _update_body(h0_ref, h1_ref, w_ref, b_ref, a0_ref, a1_ref, cv_ref, s_ref,
                 o0_ref, o1_ref):
    w = w_ref[...]
    sc = s_ref[0, 0]
    cv = cv_ref[...]
    acc = (jnp.dot(h0_ref[...], w[:_H, :], preferred_element_type=jnp.float32,
                   precision=_HIGH)
           + jnp.dot(h1_ref[...], w[_H:, :], preferred_element_type=jnp.float32,
                     precision=_HIGH)
           + b_ref[...])
    a = jnp.concatenate([a0_ref[...], a1_ref[...]], axis=1)
    acc = jnp.maximum(acc + sc * (a - cv), 0.0)
    o0_ref[...] = acc[:, :_H]
    o1_ref[...] = acc[:, _H:]


def _head_body(h0_ref, h1_ref, w1_ref, b_ref, w2_ref, o_ref):
    w1 = w1_ref[...]
    t = (jnp.dot(h0_ref[...], w1[:_H, :], preferred_element_type=jnp.float32,
                 precision=_HIGH)
         + jnp.dot(h1_ref[...], w1[_H:, :], preferred_element_type=jnp.float32,
                   precision=_HIGH)
         + b_ref[...])
    t = jnp.maximum(t, 0.0)
    o_ref[...] = jnp.dot(t, w2_ref[...], preferred_element_type=jnp.float32,
                         precision=_HIGH)


def _rows(i):
    return (i, 0)


def _rep(i):
    return (0, 0)


_SPEC_HF = pl.BlockSpec((_BM, _H), _rows)
_SPEC_FULL = pl.BlockSpec((_BM, _D), _rows)
_SPEC_W = pl.BlockSpec((_D, _D), _rep)
_SPEC_B = pl.BlockSpec((1, _D), _rep)
_SPEC_CV = pl.BlockSpec((_BM, 1), _rows)
_SPEC_S = pl.BlockSpec((1, 1), _rep)
_SPEC_W2 = pl.BlockSpec((_D, 1), _rep)
_SPEC_O1 = pl.BlockSpec((_BM, 1), _rows)

_PAIR_OUT = (jax.ShapeDtypeStruct((_N, _H), jnp.float32),
             jax.ShapeDtypeStruct((_N, _H), jnp.float32))
_PAIR_SPEC = (_SPEC_HF, _SPEC_HF)

_embed = pl.pallas_call(
    _embed_body, grid=(_GRID,),
    in_specs=[_SPEC_FULL, _SPEC_W, _SPEC_B],
    out_specs=_PAIR_SPEC, out_shape=_PAIR_OUT)

_mid = pl.pallas_call(
    _mid_body, grid=(_GRID,),
    in_specs=[_SPEC_HF, _SPEC_HF, _SPEC_W, _SPEC_B],
    out_specs=_PAIR_SPEC, out_shape=_PAIR_OUT)

_mid2 = pl.pallas_call(
    _mid2_body, grid=(_GRID,),
    in_specs=[_SPEC_HF, _SPEC_HF, _SPEC_W, _SPEC_B, _SPEC_W, _SPEC_B],
    out_specs=_PAIR_SPEC, out_shape=_PAIR_OUT)

_update = pl.pallas_call(
    _update_body, grid=(_GRID,),
    in_specs=[_SPEC_HF, _SPEC_HF, _SPEC_W, _SPEC_B, _SPEC_HF, _SPEC_HF,
              _SPEC_CV, _SPEC_S],
    out_specs=_PAIR_SPEC, out_shape=_PAIR_OUT)

_head = pl.pallas_call(
    _head_body, grid=(_GRID,),
    in_specs=[_SPEC_HF, _SPEC_HF, _SPEC_W, _SPEC_B, _SPEC_W2],
    out_specs=_SPEC_O1,
    out_shape=jax.ShapeDtypeStruct((_N, 1), jnp.float32))


def kernel(x, y, edge_index, edge_vals, c, b, W_ve, b_ve, W_ce, b_ce,
           Ukx_W, Ukx_b, Uky_W, Uky_b, tau,
           Vky_W, Vky_b, Wkx_W, Wkx_b, Vkx_W, Vkx_b, sigma,
           W_o1, b_o1, W_o2):
    L = Ukx_W.shape[0]
    rows = edge_index[0].astype(jnp.int32)
    cols = edge_index[1].astype(jnp.int32)
    vals = edge_vals.astype(jnp.float32)

    E = vals.shape[0]
    nchunk = -(-E // (_NS * _SB * _CH)) * _SB
    e_pad = nchunk * _CH * _NS
    if e_pad != E:
        pz = e_pad - E
        rows = jnp.concatenate([rows, jnp.zeros((pz,), jnp.int32)])
        cols = jnp.concatenate([cols, jnp.zeros((pz,), jnp.int32)])
        vals = jnp.concatenate([vals, jnp.zeros((pz,), jnp.float32)])
    rows = rows.reshape(_NS, nchunk, _CH)
    cols = cols.reshape(_NS, nchunk, _CH)
    vals = vals.reshape(_NS, nchunk, _CH)

    b_ve2 = b_ve.reshape(1, _D)
    b_ce2 = b_ce.reshape(1, _D)
    b_o12 = b_o1.reshape(1, _D)

    x0, x1 = _embed(x, W_ve, b_ve2)
    y0, y1 = _embed(y, W_ce, b_ce2)

    for l in range(L):
        u0, u1 = _mid(y0, y1, Uky_W[l], Uky_b[l].reshape(1, _D))
        a0, a1 = _spmm(u0, u1, rows, cols, vals, nchunk)
        x0, x1 = _update(x0, x1, Ukx_W[l], Ukx_b[l].reshape(1, _D),
                         a0, a1, c, tau[l].reshape(1, 1))
        z0, z1 = _mid2(x0, x1, Wkx_W[l], Wkx_b[l].reshape(1, _D),
                       Vkx_W[l], Vkx_b[l].reshape(1, _D))
        s0, s1 = _spmm(z0, z1, cols, rows, vals, nchunk)
        y0, y1 = _update(y0, y1, Vky_W[l], Vky_b[l].reshape(1, _D),
                         s0, s1, b, sigma[l].reshape(1, 1))

    x_out = _head(x0, x1, W_o1, b_o12, W_o2).reshape(_N)
    y_out = _head(y0, y1, W_o1, b_o12, W_o2).reshape(_N)
    return (x_out, y_out)
